# Initial kernel scaffold; baseline (speedup 1.0000x reference)
import jax
import jax.numpy as jnp
from jax.experimental import pallas as pl


def kernel(embeddings1, embeddings2, overlap_ratio):
    raise NotImplementedError("write your pallas kernel here")


# speedup vs baseline: 3.7172x; 3.7172x over previous
import jax
import jax.numpy as jnp
from jax import lax
from jax.experimental import pallas as pl
from jax.experimental.pallas import tpu as pltpu

_BASE_MARGIN = 0.5
_POS_THR = 0.7
_NEG_THR = 0.2
_B = 256
_CHUNK = 8
_BIG = 1e30


def _triplet_body(e1_ref, e2t_ref, ov_ref, out_ref):
    e1 = e1_ref[...]
    e2t = e2t_ref[...]
    ov = ov_ref[...]

    n1 = jnp.sqrt(jnp.sum(e1 * e1, axis=1, keepdims=True))
    e1n = e1 / jnp.maximum(n1, 1e-12)
    n2 = jnp.sqrt(jnp.sum(e2t * e2t, axis=0, keepdims=True))
    e2nt = e2t / jnp.maximum(n2, 1e-12)

    s1 = jnp.sum(e1n * e1n, axis=1, keepdims=True)
    s2 = jnp.sum(e2nt * e2nt, axis=0, keepdims=True)
    g = jnp.dot(e1n, e2nt, preferred_element_type=jnp.float32)
    d = jnp.sqrt(jnp.maximum(s1 + s2 - 2.0 * g, 1e-12))

    z = d + _BASE_MARGIN * ov
    pos = ov > _POS_THR
    neg = ov <= _NEG_THR
    zp = jnp.where(pos, z, -_BIG)
    zn = jnp.where(neg, z, _BIG)

    cp = jnp.sum(pos.astype(jnp.float32), axis=1, keepdims=True)
    cn = jnp.sum(neg.astype(jnp.float32), axis=1, keepdims=True)
    count = jnp.sum(cp * cn)

    acc = jnp.zeros((_CHUNK, _B), jnp.float32)
    for i in range(_B // _CHUNK):
        zp_c = zp[i * _CHUNK:(i + 1) * _CHUNK, :]
        zn_c = zn[i * _CHUNK:(i + 1) * _CHUNK, :]
        t = jnp.maximum(zp_c[:, :, None] - zn_c[:, None, :], 0.0)
        acc = acc + jnp.sum(t, axis=1)
    total = jnp.sum(acc)
    out_ref[0, 0] = jnp.where(count == 0.0, jnp.float32(0.0),
                              total / jnp.maximum(count, 1.0))


def kernel(embeddings1, embeddings2, overlap_ratio):
    out = pl.pallas_call(
        _triplet_body,
        out_shape=jax.ShapeDtypeStruct((1, 1), jnp.float32),
        out_specs=pl.BlockSpec(memory_space=pltpu.SMEM),
    )(embeddings1, embeddings2.T, overlap_ratio)
    return jnp.reshape(out, ())
